# baseline (speedup 1.0000x reference)
import functools

import jax
import jax.numpy as jnp
from jax import lax
from jax.experimental import pallas as pl
from jax.experimental.pallas import tpu as pltpu
from jax.experimental.pallas import tpu_sc as plsc

B = 16384
V = 100000
D = 16
N_CAT = 4

NC = 2
NS = 16
L = 16
NW = NC * NS
BPW = B // NW
CH = 128
NCH = BPW // CH

_mesh = plsc.VectorSubcoreMesh(core_axis_name="c", subcore_axis_name="s")


@functools.partial(
    pl.kernel,
    out_type=(
        jax.ShapeDtypeStruct((B * 2,), jnp.float32),
        jax.ShapeDtypeStruct((B, D), jnp.float32),
        jax.ShapeDtypeStruct((B, D), jnp.float32),
        jax.ShapeDtypeStruct((B, D), jnp.float32),
        jax.ShapeDtypeStruct((B, D), jnp.float32),
    ),
    mesh=_mesh,
    compiler_params=pltpu.CompilerParams(
        needs_layout_passes=False, use_tc_tiling_on_sc=False),
    scratch_types=[
        pltpu.VMEM((N_CAT, NCH, CH), jnp.int32),
        pltpu.VMEM((N_CAT, BPW, D), jnp.float32),
        pltpu.VMEM((BPW,), jnp.float32),
        pltpu.VMEM((BPW,), jnp.float32),
        pltpu.VMEM((BPW * 2,), jnp.float32),
        pltpu.SemaphoreType.DMA,
    ],
)
def _input_layer_sc(n0_hbm, n1_hbm, t_all, c0, c1, c2, c3,
                    out_num, out0, out1, out2, out3,
                    idx_v, rows_v, n0_v, n1_v, nbuf, sem):
    wid = lax.axis_index("s") * NC + lax.axis_index("c")
    base = wid * BPW
    tables = tuple(t_all.at[t] for t in range(N_CAT))
    cats = (c0, c1, c2, c3)
    outs = (out0, out1, out2, out3)

    for t in range(N_CAT):
        pltpu.sync_copy(cats[t].at[wid], idx_v.at[t])
    pltpu.sync_copy(n0_hbm.at[wid], n0_v)
    pltpu.sync_copy(n1_hbm.at[wid], n1_v)

    copies = []
    for t in range(N_CAT):
        for j in range(NCH):
            copies.append(
                pltpu.async_copy(
                    tables[t].at[idx_v.at[t, j]],
                    rows_v.at[t, pl.ds(j * CH, CH)],
                    sem,
                )
            )

    lane = lax.iota(jnp.int32, L)
    for i in range(BPW // L):
        flat = (lane + i * L) * 2
        v0 = n0_v[pl.ds(i * L, L)]
        v1 = n1_v[pl.ds(i * L, L)]
        plsc.store_scatter(nbuf, [flat], v0)
        plsc.store_scatter(nbuf, [flat + 1], v1)
    pltpu.sync_copy(nbuf, out_num.at[pl.ds(base * 2, BPW * 2)])

    for c in copies:
        c.wait()
    for t in range(N_CAT):
        pltpu.sync_copy(rows_v.at[t], outs[t].at[pl.ds(base, BPW)])


def kernel(num_0, num_1, emb_cat_0, emb_cat_1, emb_cat_2, emb_cat_3,
           cat_0, cat_1, cat_2, cat_3):
    n0 = num_0.astype(jnp.float32).reshape(NW, BPW)
    n1 = num_1.astype(jnp.float32).reshape(NW, BPW)
    c0 = cat_0.reshape(NW, NCH, CH)
    c1 = cat_1.reshape(NW, NCH, CH)
    c2 = cat_2.reshape(NW, NCH, CH)
    c3 = cat_3.reshape(NW, NCH, CH)
    t_all = jnp.stack([emb_cat_0, emb_cat_1, emb_cat_2, emb_cat_3])
    out_num, e0, e1, e2, e3 = _input_layer_sc(
        n0, n1, t_all, c0, c1, c2, c3)
    return (out_num.reshape(B, 2), e0, e1, e2, e3)


# speedup vs baseline: 1.1151x; 1.1151x over previous
import functools

import jax
import jax.numpy as jnp
from jax import lax
from jax.experimental import pallas as pl
from jax.experimental.pallas import tpu as pltpu
from jax.experimental.pallas import tpu_sc as plsc

B = 16384
V = 100000
D = 16
N_CAT = 4

NC = 2
NS = 16
L = 16
NW = NC * NS
BPW = B // NW
CH = 128
NCH = BPW // CH

_mesh = plsc.VectorSubcoreMesh(core_axis_name="c", subcore_axis_name="s")
_params = pltpu.CompilerParams(
    needs_layout_passes=False, use_tc_tiling_on_sc=False)


@functools.partial(
    pl.kernel,
    out_type=jax.ShapeDtypeStruct((B, D), jnp.float32),
    mesh=_mesh,
    compiler_params=_params,
    scratch_types=[
        pltpu.VMEM((NCH, CH), jnp.int32),
        pltpu.VMEM((BPW, D), jnp.float32),
        pltpu.SemaphoreType.DMA,
    ],
)
def _gather_sc(table, cat, out, idx_v, rows_v, sem):
    wid = lax.axis_index("s") * NC + lax.axis_index("c")
    base = wid * BPW
    pltpu.sync_copy(cat.at[wid], idx_v)
    copies = []
    for j in range(NCH):
        copies.append(
            pltpu.async_copy(
                table.at[idx_v.at[j]],
                rows_v.at[pl.ds(j * CH, CH)],
                sem,
            )
        )
    for c in copies:
        c.wait()
    pltpu.sync_copy(rows_v, out.at[pl.ds(base, BPW)])


@functools.partial(
    pl.kernel,
    out_type=jax.ShapeDtypeStruct((B * 2,), jnp.float32),
    mesh=_mesh,
    compiler_params=_params,
    scratch_types=[
        pltpu.VMEM((BPW,), jnp.float32),
        pltpu.VMEM((BPW,), jnp.float32),
        pltpu.VMEM((BPW * 2,), jnp.float32),
    ],
)
def _concat_sc(n0_hbm, n1_hbm, out_num, n0_v, n1_v, nbuf):
    wid = lax.axis_index("s") * NC + lax.axis_index("c")
    base = wid * BPW
    pltpu.sync_copy(n0_hbm.at[wid], n0_v)
    pltpu.sync_copy(n1_hbm.at[wid], n1_v)
    lane = lax.iota(jnp.int32, L)
    for i in range(BPW // L):
        flat = (lane + i * L) * 2
        v0 = n0_v[pl.ds(i * L, L)]
        v1 = n1_v[pl.ds(i * L, L)]
        plsc.store_scatter(nbuf, [flat], v0)
        plsc.store_scatter(nbuf, [flat + 1], v1)
    pltpu.sync_copy(nbuf, out_num.at[pl.ds(base * 2, BPW * 2)])


def kernel(num_0, num_1, emb_cat_0, emb_cat_1, emb_cat_2, emb_cat_3,
           cat_0, cat_1, cat_2, cat_3):
    n0 = num_0.astype(jnp.float32).reshape(NW, BPW)
    n1 = num_1.astype(jnp.float32).reshape(NW, BPW)
    out_num = _concat_sc(n0, n1)
    es = []
    for tbl, cat in ((emb_cat_0, cat_0), (emb_cat_1, cat_1),
                     (emb_cat_2, cat_2), (emb_cat_3, cat_3)):
        es.append(_gather_sc(tbl, cat.reshape(NW, NCH, CH)))
    return (out_num.reshape(B, 2), es[0], es[1], es[2], es[3])
